# baseline (speedup 1.0000x reference)
import functools

import jax
import jax.numpy as jnp
from jax import lax
from jax.experimental import pallas as pl
from jax.experimental.pallas import tpu as pltpu


def _round_up(a, b):
    return ((a + b - 1) // b) * b


def _vocab_sweep_kernel(x_ref, w_ref, o_ref):
    m_tot = x_ref.shape[0]
    for lo in range(0, m_tot, 1024):
        hi = min(lo + 1024, m_tot)
        o_ref[lo:hi, :] = lax.dot_general(
            x_ref[lo:hi, :], w_ref[...],
            dimension_numbers=(((1,), (1,)), ((), ())),
            preferred_element_type=jnp.float32,
        ).astype(o_ref.dtype)


def _pick_tn(vocab, k_pad, m_pad, resident_bytes, budget):
    def fits(t):
        return resident_bytes + 2 * (4 * t * k_pad) + 2 * (4 * m_pad * t) <= budget // 2

    for t in (512, 1024):
        if t <= vocab and fits(t):
            return t
    for t in (256, 128):
        if t <= vocab and fits(t):
            return t
    return 128


def kernel(x, weight):
    *lead, hidden = x.shape
    vocab, hidden_w = weight.shape
    assert hidden == hidden_w, "hidden size mismatch between x and weight"
    out_dtype = x.dtype

    m = 1
    for d in lead:
        m *= d

    k_pad = _round_up(hidden, 128)
    m_pad = _round_up(max(m, 1), 16)

    x2d = x.reshape(m, hidden)
    if (m_pad, k_pad) != (m, hidden):
        x2d = jnp.pad(x2d, ((0, m_pad - m), (0, k_pad - hidden)))
    w2d = weight
    if k_pad != hidden:
        w2d = jnp.pad(w2d, ((0, 0), (0, k_pad - hidden)))

    budget = 64 * 1024 * 1024
    resident = 4 * m_pad * k_pad

    if resident <= budget // 2:
        tn = _pick_tn(vocab, k_pad, m_pad, resident, budget)
        n_n = pl.cdiv(vocab, tn)
        cost = pl.CostEstimate(
            flops=2 * m * vocab * hidden,
            transcendentals=0,
            bytes_accessed=(x2d.size * 4 + w2d.size * 4 + m_pad * vocab * 4),
        )
        out2d = pl.pallas_call(
            _vocab_sweep_kernel,
            out_shape=jax.ShapeDtypeStruct((m_pad, vocab), out_dtype),
            grid=(n_n,),
            in_specs=[
                pl.BlockSpec((m_pad, k_pad), lambda j: (0, 0)),
                pl.BlockSpec((tn, k_pad), lambda j: (j, 0)),
            ],
            out_specs=pl.BlockSpec((m_pad, tn), lambda j: (0, j)),
            compiler_params=pltpu.CompilerParams(
                dimension_semantics=("parallel",),
                vmem_limit_bytes=budget,
            ),
            cost_estimate=cost,
        )(x2d, w2d)
    else:
        tm = next(t for t in (2048, 1024, 512, 256, 128, 64, 32, 16)
                  if 4 * t * k_pad <= budget // 4)
        tn = _pick_tn(vocab, k_pad, tm, 4 * tm * k_pad, budget)
        n_m = pl.cdiv(m_pad, tm)
        n_n = pl.cdiv(vocab, tn)
        cost = pl.CostEstimate(
            flops=2 * m * vocab * hidden,
            transcendentals=0,
            bytes_accessed=(x2d.size * 4 + n_m * w2d.size * 4
                            + m_pad * vocab * 4),
        )
        out2d = pl.pallas_call(
            _vocab_sweep_kernel,
            out_shape=jax.ShapeDtypeStruct((m_pad, vocab), out_dtype),
            grid=(n_m, n_n),
            in_specs=[
                pl.BlockSpec((tm, k_pad), lambda i, j: (i, 0)),
                pl.BlockSpec((tn, k_pad), lambda i, j: (j, 0)),
            ],
            out_specs=pl.BlockSpec((tm, tn), lambda i, j: (i, j)),
            compiler_params=pltpu.CompilerParams(
                dimension_semantics=("parallel", "parallel"),
                vmem_limit_bytes=budget,
            ),
            cost_estimate=cost,
        )(x2d, w2d)

    if m_pad != m:
        out2d = out2d[:m]
    return out2d.reshape(*lead, vocab)


# speedup vs baseline: 1.0087x; 1.0087x over previous
import functools

import jax
import jax.numpy as jnp
from jax import lax
from jax.experimental import pallas as pl
from jax.experimental.pallas import tpu as pltpu


def _round_up(a, b):
    return ((a + b - 1) // b) * b


def _vocab_sweep_kernel(x_ref, w_ref, o_ref, *, vocab, n_axis):
    m_tot = x_ref.shape[0]
    tn = w_ref.shape[0]
    for nc in range(0, tn, 512):
        nhi = min(nc + 512, tn)

        def _do(nc=nc, nhi=nhi):
            for lo in range(0, m_tot, 1024):
                hi = min(lo + 1024, m_tot)
                o_ref[lo:hi, nc:nhi] = lax.dot_general(
                    x_ref[lo:hi, :], w_ref[nc:nhi, :],
                    dimension_numbers=(((1,), (1,)), ((), ())),
                    preferred_element_type=jnp.float32,
                ).astype(o_ref.dtype)

        if nc == 0:
            _do()
        else:
            pl.when(pl.program_id(n_axis) * tn + nc < vocab)(_do)


def _pick_tn(vocab, k_pad, m_pad, resident_bytes, budget):
    def fits(t):
        return resident_bytes + 2 * (4 * t * k_pad) + 2 * (4 * m_pad * t) <= (52 * 1024 * 1024)

    for t in (1024, 512):
        if t <= vocab and fits(t):
            return t
    for t in (256, 128):
        if t <= vocab and fits(t):
            return t
    return 128


def kernel(x, weight):
    *lead, hidden = x.shape
    vocab, hidden_w = weight.shape
    assert hidden == hidden_w, "hidden size mismatch between x and weight"
    out_dtype = x.dtype

    m = 1
    for d in lead:
        m *= d

    k_pad = _round_up(hidden, 128)
    m_pad = _round_up(max(m, 1), 16)

    x2d = x.reshape(m, hidden)
    if (m_pad, k_pad) != (m, hidden):
        x2d = jnp.pad(x2d, ((0, m_pad - m), (0, k_pad - hidden)))
    w2d = weight
    if k_pad != hidden:
        w2d = jnp.pad(w2d, ((0, 0), (0, k_pad - hidden)))

    budget = 64 * 1024 * 1024
    resident = 4 * m_pad * k_pad

    if resident <= budget // 2:
        tn = _pick_tn(vocab, k_pad, m_pad, resident, budget)
        n_n = pl.cdiv(vocab, tn)
        cost = pl.CostEstimate(
            flops=2 * m * vocab * hidden,
            transcendentals=0,
            bytes_accessed=(x2d.size * 4 + w2d.size * 4 + m_pad * vocab * 4),
        )
        out2d = pl.pallas_call(
            functools.partial(_vocab_sweep_kernel, vocab=vocab, n_axis=0),
            out_shape=jax.ShapeDtypeStruct((m_pad, vocab), out_dtype),
            grid=(n_n,),
            in_specs=[
                pl.BlockSpec((m_pad, k_pad), lambda j: (0, 0)),
                pl.BlockSpec((tn, k_pad), lambda j: (j, 0)),
            ],
            out_specs=pl.BlockSpec((m_pad, tn), lambda j: (0, j)),
            compiler_params=pltpu.CompilerParams(
                dimension_semantics=("parallel",),
                vmem_limit_bytes=budget,
            ),
            cost_estimate=cost,
        )(x2d, w2d)
    else:
        tm = next(t for t in (2048, 1024, 512, 256, 128, 64, 32, 16)
                  if 4 * t * k_pad <= budget // 4)
        tn = _pick_tn(vocab, k_pad, tm, 4 * tm * k_pad, budget)
        n_m = pl.cdiv(m_pad, tm)
        n_n = pl.cdiv(vocab, tn)
        cost = pl.CostEstimate(
            flops=2 * m * vocab * hidden,
            transcendentals=0,
            bytes_accessed=(x2d.size * 4 + n_m * w2d.size * 4
                            + m_pad * vocab * 4),
        )
        out2d = pl.pallas_call(
            functools.partial(_vocab_sweep_kernel, vocab=vocab, n_axis=1),
            out_shape=jax.ShapeDtypeStruct((m_pad, vocab), out_dtype),
            grid=(n_m, n_n),
            in_specs=[
                pl.BlockSpec((tm, k_pad), lambda i, j: (i, 0)),
                pl.BlockSpec((tn, k_pad), lambda i, j: (j, 0)),
            ],
            out_specs=pl.BlockSpec((tm, tn), lambda i, j: (i, j)),
            compiler_params=pltpu.CompilerParams(
                dimension_semantics=("parallel", "parallel"),
                vmem_limit_bytes=budget,
            ),
            cost_estimate=cost,
        )(x2d, w2d)

    if m_pad != m:
        out2d = out2d[:m]
    return out2d.reshape(*lead, vocab)
